# Initial kernel scaffold; baseline (speedup 1.0000x reference)
import jax
import jax.numpy as jnp
from jax.experimental import pallas as pl


def kernel(users, items, user_table, item_table, symptom_table, herb_table, edge_src, edge_dst, edge_w, user_layer_weights, item_layer_weights):
    raise NotImplementedError("write your pallas kernel here")


# speedup vs baseline: 1.2252x; 1.2252x over previous
import jax
import jax.numpy as jnp
from jax import lax
from jax.experimental import pallas as pl
from jax.experimental.pallas import tpu as pltpu
from jax.experimental.pallas import tpu_sc as plsc

NU = 25000
NI = 25000
SEG = 25088
NP2 = 2 * SEG
D = 64
H = 32
W = 128
E = 800000
K = 32
CH = 25088
EP = CH * K
NLAY = 3
B = 4096
RB = 32
BLK_PER_TILE = NP2 // RB // 16
CH_PER_TILE = CH // 16
ROWS_PER_TILE = NP2 // 16


def _sc_body(base_h, sem_h, io_t_h, es_h, ed_h, ew_h, ur_h, ir_h, ab_h,
             lay0_h, lay1_h, lay2_h, lay3_h, uo_h, io_h,
             acc_sh, es_v, ed_v, ew_v, rows_v, msg_v, blk_a, blk_b, blk_c,
             zbuf, ab_v, up_v, ip_v, ix_v, sem1, sem2, sem3):
    lays = [lay0_h, lay1_h, lay2_h, lay3_h]
    c = lax.axis_index("c")
    s = lax.axis_index("s")
    pltpu.sync_copy(ab_h, ab_v)

    z16 = jnp.zeros((16,), jnp.float32)
    @pl.loop(0, RB)
    def _(i):
        zbuf[i, pl.ds(0, 16)] = z16
        zbuf[i, pl.ds(16, 16)] = z16
        for d in range(2, 8):
            blk_a[i, pl.ds(d * 16, 16)] = z16

    r_base = s * ROWS_PER_TILE
    in_item = s >= 8

    def fill_ix(r0):
        pltpu.sync_copy(io_t_h.at[pl.ds(r0, RB)], ix_v)

    def blend(layer):
        av = jnp.where(in_item, ab_v[1, layer, pl.ds(0, 16)],
                       ab_v[0, layer, pl.ds(0, 16)])
        out = lays[layer]
        @pl.loop(0, BLK_PER_TILE)
        def _(b):
            r0 = r_base + b * RB
            pltpu.sync_copy(sem_h.at[pl.ds(c * NP2 + r0, RB)], blk_b)
            if layer == 0:
                pltpu.sync_copy(base_h.at[pl.ds(c * NP2 + r0, RB)], blk_c)
            else:
                fill_ix(r0)
                pltpu.sync_copy(acc_sh.at[ix_v], blk_c)
            @pl.loop(0, RB)
            def _(i):
                for d in range(2):
                    sl = (i, pl.ds(d * 16, 16))
                    blk_a[sl] = av * blk_c[sl] + (blk_b[sl] - av * blk_b[sl])
            pltpu.sync_copy(blk_a, out.at[pl.ds(c * NP2 + r0, RB)])

    blend(0)
    plsc.subcore_barrier()

    for layer in range(1, NLAY + 1):
        @pl.loop(0, BLK_PER_TILE)
        def _(b):
            fill_ix(r_base + b * RB)
            pltpu.sync_copy(acc_sh.at[ix_v], blk_c)
            @pl.loop(0, RB)
            def _(i):
                for d in range(2):
                    sl = (i, pl.ds(d * 16, 16))
                    blk_c[sl] = z16 - blk_c[sl]
            pltpu.sync_copy(blk_c, acc_sh.at[ix_v], add=True)
        plsc.subcore_barrier()

        prev = lays[layer - 1]
        c0 = s * CH_PER_TILE
        @pl.loop(0, CH_PER_TILE)
        def _(ci):
            e0 = (c0 + ci) * K
            cp1 = pltpu.async_copy(es_h.at[c, pl.ds(e0, K)], es_v, sem1)
            cp2 = pltpu.async_copy(ed_h.at[pl.ds(e0, K)], ed_v, sem2)
            cp3 = pltpu.async_copy(ew_h.at[pl.ds(e0, K)], ew_v, sem3)
            cp1.wait()
            cp2.wait()
            cp3.wait()
            pltpu.sync_copy(prev.at[es_v], rows_v)
            @pl.loop(0, K)
            def _(i):
                for d in range(2):
                    sl = (i, pl.ds(d * 16, 16))
                    msg_v[sl] = rows_v[sl] * ew_v[sl]
            pltpu.sync_copy(msg_v, acc_sh.at[ed_v], add=True)
        plsc.subcore_barrier()
        blend(layer)
        plsc.subcore_barrier()

    p0 = s * (B // 16)
    for pc in range(B // 16 // K):
        q0 = p0 + pc * K
        cpu = pltpu.async_copy(ur_h.at[c, pl.ds(q0, K)], up_v, sem1)
        cpi = pltpu.async_copy(ir_h.at[c, pl.ds(q0, K)], ip_v, sem2)
        cpu.wait()
        cpi.wait()

        def accum(idx_v, dst_h):
            pltpu.sync_copy(lays[0].at[idx_v], rows_v)
            @pl.loop(0, K)
            def _(i):
                for d in range(2):
                    sl = (i, pl.ds(d * 16, 16))
                    msg_v[sl] = rows_v[sl]
            for layer in range(1, NLAY + 1):
                pltpu.sync_copy(lays[layer].at[idx_v], rows_v)
                @pl.loop(0, K)
                def _(i):
                    for d in range(2):
                        sl = (i, pl.ds(d * 16, 16))
                        msg_v[sl] = msg_v[sl] + rows_v[sl]
            pltpu.sync_copy(msg_v, dst_h.at[pl.ds(c * B + q0, K)])

        accum(up_v, uo_h)
        accum(ip_v, io_h)


def _combine_body(u_ref, i_ref, o_ref):
    p = u_ref[...] * i_ref[...]
    ps = jnp.sum(p, axis=1, keepdims=True)
    g = (ps[:B] + ps[B:]) * 0.0625
    o_ref[...] = jnp.broadcast_to(g, (B, 128))


def kernel(users, items, user_table, item_table, symptom_table, herb_table,
           edge_src, edge_dst, edge_w, user_layer_weights, item_layer_weights):
    f32 = jnp.float32
    i32 = jnp.int32

    sig_u = jax.nn.sigmoid(user_layer_weights.astype(f32))
    sig_i = jax.nn.sigmoid(item_layer_weights.astype(f32))
    ab = jnp.stack([jnp.broadcast_to(sig_u[:, None], (NLAY + 1, 16)),
                    jnp.broadcast_to(sig_i[:, None], (NLAY + 1, 16))])

    def halves(u_tbl, i_tbl):
        parts = []
        for cdx in range(2):
            uh = jnp.pad(u_tbl[:, cdx * H:(cdx + 1) * H], ((0, SEG - NU), (0, 0)))
            ih = jnp.pad(i_tbl[:, cdx * H:(cdx + 1) * H], ((0, SEG - NI), (0, 0)))
            parts += [uh, ih]
        return jnp.concatenate(parts, axis=0).astype(f32)

    base = halves(user_table, item_table)
    sem_t = halves(symptom_table, herb_table)
    iota_t = jnp.arange(NP2, dtype=i32)

    off = SEG - NU
    es = jnp.where(edge_src >= NU, edge_src + off, edge_src).astype(i32)
    ed = jnp.where(edge_dst >= NU, edge_dst + off, edge_dst).astype(i32)
    es = jnp.pad(es, (0, EP - E))
    ed = jnp.pad(ed, (0, EP - E))
    ew1 = jnp.pad(edge_w.astype(f32), (0, EP - E))
    ew = jnp.broadcast_to(ew1[:, None], (EP, H))
    es2 = jnp.stack([es, es + NP2])
    ur = users.astype(i32)
    ir = items.astype(i32) + SEG
    ur2 = jnp.stack([ur, ur + NP2])
    ir2 = jnp.stack([ir, ir + NP2])

    mesh = plsc.VectorSubcoreMesh(core_axis_name="c", subcore_axis_name="s",
                                  num_cores=2, num_subcores=16)
    buf = jax.ShapeDtypeStruct((2 * NP2, W), f32)
    pair = jax.ShapeDtypeStruct((2 * B, H), f32)
    out_types = [buf, buf, buf, buf, pair, pair]
    scratch = [
        pltpu.VMEM_SHARED((NP2, H), f32),
        pltpu.VMEM((K,), i32),
        pltpu.VMEM((K,), i32),
        pltpu.VMEM((K, H), f32),
        pltpu.VMEM((K, W), f32),
        pltpu.VMEM((K, H), f32),
        pltpu.VMEM((RB, W), f32),
        pltpu.VMEM((RB, H), f32),
        pltpu.VMEM((RB, H), f32),
        pltpu.VMEM((RB, H), f32),
        pltpu.VMEM((2, NLAY + 1, 16), f32),
        pltpu.VMEM((K,), i32),
        pltpu.VMEM((K,), i32),
        pltpu.VMEM((RB,), i32),
        pltpu.SemaphoreType.DMA,
        pltpu.SemaphoreType.DMA,
        pltpu.SemaphoreType.DMA,
    ]
    sc_fn = pl.kernel(_sc_body, out_type=out_types, mesh=mesh,
                      scratch_types=scratch)
    _, _, _, _, uo, io_ = sc_fn(base, sem_t, iota_t, es2, ed, ew, ur2, ir2, ab)

    gout = pl.pallas_call(
        _combine_body,
        out_shape=jax.ShapeDtypeStruct((B, 128), f32),
    )(uo, io_)
    return gout[:, 0]
